# baseline (speedup 1.0000x reference)
import functools

import jax
import jax.numpy as jnp
from jax import lax
from jax.experimental import pallas as pl
from jax.experimental.pallas import tpu as pltpu
from jax.experimental.pallas import tpu_sc as plsc

N = 10000
D = 128
E = 320000
HEADS = 8
HD = 16
NW = 32
EPW = E // NW
CHUNK = 40
NBLK = EPW // CHUNK
NDEN = 632
SB = 400
KB = SB // CHUNK
NSB = EPW // SB


_BM = 1000


def _proj_body(x_ref, w_ref, b_ref, o_ref):
    o_ref[...] = (
        jnp.dot(x_ref[...], w_ref[...], preferred_element_type=jnp.float32)
        + b_ref[...]
    )


def _proj(x, w, b):
    return pl.pallas_call(
        _proj_body,
        grid=(N // _BM,),
        in_specs=[
            pl.BlockSpec((_BM, D), lambda i: (i, 0)),
            pl.BlockSpec((D, D), lambda i: (0, 0)),
            pl.BlockSpec((1, D), lambda i: (0, 0)),
        ],
        out_specs=pl.BlockSpec((_BM, D), lambda i: (i, 0)),
        out_shape=jax.ShapeDtypeStruct((N, D), jnp.float32),
    )(x, w, b.reshape(1, D))


def _qkv_body(h_ref, wkv_ref, wq_ref, kv_ref, q_ref):
    h = h_ref[...]
    kv_ref[...] = jnp.dot(h, wkv_ref[...], preferred_element_type=jnp.float32)
    q_ref[...] = jnp.dot(h, wq_ref[...], preferred_element_type=jnp.float32)


def _qkv(h, wkv, wq):
    return pl.pallas_call(
        _qkv_body,
        grid=(N // _BM,),
        in_specs=[
            pl.BlockSpec((_BM, D), lambda i: (i, 0)),
            pl.BlockSpec((D, 2 * D), lambda i: (0, 0)),
            pl.BlockSpec((D, D), lambda i: (0, 0)),
        ],
        out_specs=[
            pl.BlockSpec((_BM, 2 * D), lambda i: (i, 0)),
            pl.BlockSpec((_BM, D), lambda i: (i, 0)),
        ],
        out_shape=[
            jax.ShapeDtypeStruct((N, 2 * D), jnp.float32),
            jax.ShapeDtypeStruct((N, D), jnp.float32),
        ],
    )(h, wkv, wq)


def _layer_norm(x, g, b):
    mu = jnp.mean(x, axis=-1, keepdims=True)
    xc = x - mu
    var = jnp.mean(xc * xc, axis=-1, keepdims=True)
    return xc * lax.rsqrt(var + 1e-5) * g + b


def _post_body(num_ref, den_ref, h_ref, exp_ref, wo_ref, bo_ref, g1_ref,
               b1_ref, w1_ref, bf1_ref, w2_ref, bf2_ref, g2_ref, b2_ref,
               o_ref):
    numer = num_ref[0] + num_ref[1]
    den8 = den_ref[0] + den_ref[1]
    dd = jnp.dot(den8, exp_ref[...], preferred_element_type=jnp.float32)
    att = numer / (dd + 1e-6)
    y = (
        jnp.dot(att, wo_ref[...], preferred_element_type=jnp.float32)
        + bo_ref[...]
        + h_ref[...]
    )
    y = _layer_norm(y, g1_ref[...], b1_ref[...])
    f = jnp.maximum(
        jnp.dot(y, w1_ref[...], preferred_element_type=jnp.float32)
        + bf1_ref[...],
        0.0,
    )
    f = jnp.dot(f, w2_ref[...], preferred_element_type=jnp.float32) + bf2_ref[...]
    z = y + f
    o_ref[...] = _layer_norm(z, g2_ref[...], b2_ref[...])


def _post(num, den, h, lp, expand):
    row = lambda v: v.reshape(1, -1)
    full = lambda shp: pl.BlockSpec(shp, lambda i: (0,) * len(shp))
    return pl.pallas_call(
        _post_body,
        grid=(N // _BM,),
        in_specs=[
            pl.BlockSpec((2, _BM, D), lambda i: (0, i, 0)),
            pl.BlockSpec((2, _BM, HEADS), lambda i: (0, i, 0)),
            pl.BlockSpec((_BM, D), lambda i: (i, 0)),
            full((HEADS, D)),
            full((D, D)),
            full((1, D)),
            full((1, D)),
            full((1, D)),
            full((D, 2 * D)),
            full((1, 2 * D)),
            full((2 * D, D)),
            full((1, D)),
            full((1, D)),
            full((1, D)),
        ],
        out_specs=pl.BlockSpec((_BM, D), lambda i: (i, 0)),
        out_shape=jax.ShapeDtypeStruct((N, D), jnp.float32),
    )(
        num, den, h, expand, lp['WO'], row(lp['bO']), row(lp['ln1_g']),
        row(lp['ln1_b']), lp['W1'], row(lp['b1']), lp['W2'], row(lp['b2']),
        row(lp['ln2_g']), row(lp['ln2_b']),
    )


def _edge_body(kv_hbm, q_hbm, src_hbm, dst_hbm, dst16_hbm,
               zero_hbm, num_hbm, den_hbm,
               src_big, dstv_big, dsc0, dsc1, d16sc0, d16sc1,
               kv0, kv1, q0, q1, out_rows, den_rows, acc_sh, den_sh,
               gsem0, gsem1, ssem, isem0, isem1):
    c = lax.axis_index("c")
    s = lax.axis_index("s")

    @pl.when(s == 0)
    def _():
        pltpu.sync_copy(zero_hbm, acc_sh)
        pltpu.sync_copy(zero_hbm.at[pl.ds(0, NDEN)], den_sh)

    plsc.subcore_barrier()

    lane_ids = lax.iota(jnp.int32, 16)
    lane_ge8 = (lane_ids >= 8).astype(jnp.int32)
    lane_mod8 = lane_ids & 7
    zeros16i = jnp.zeros((16,), jnp.int32)
    shuf_idx = [lane_ids ^ k for k in (8, 4, 2, 1)]

    def allsum16(v):
        for idx in shuf_idx:
            v = v + v.at[idx].get(mode='promise_in_bounds')
        return v

    kvb = (kv0, kv1)
    qb = (q0, q1)
    dscb = (dsc0, dsc1)
    d16b = (d16sc0, d16sc1)
    gsem = (gsem0, gsem1)
    isem = (isem0, isem1)
    wid = c * 16 + s
    base_w = wid * EPW

    def sblk(sb, carry):
        base = base_w + sb * SB
        pltpu.sync_copy(src_hbm.at[pl.ds(base, SB)], src_big)
        pltpu.sync_copy(dst_hbm.at[pl.ds(base, SB)], dstv_big.at[pl.ds(0, SB)])

        def grab(k, b):
            off = pl.multiple_of(k * CHUNK, 8)
            for t in range(0, CHUNK, 8):
                i1 = src_big.at[pl.ds(off + t, 8)]
                i2 = dstv_big.at[pl.ds(off + t, 8)]
                pltpu.async_copy(kv_hbm.at[i1], kvb[b].at[pl.ds(t, 8)],
                                 gsem[b])
                pltpu.async_copy(q_hbm.at[i2], qb[b].at[pl.ds(t, 8)],
                                 gsem[b])

        def wait_grab(b):
            pltpu.make_async_copy(
                kv_hbm.at[pl.ds(0, CHUNK)], kvb[b], gsem[b]).wait()
            pltpu.make_async_copy(
                q_hbm.at[pl.ds(0, CHUNK)], qb[b], gsem[b]).wait()

        def grab_idx(k, b):
            b0 = base + k * CHUNK
            pltpu.async_copy(dst_hbm.at[pl.ds(b0, CHUNK)], dscb[b], isem[b])
            pltpu.async_copy(dst16_hbm.at[pl.ds(b0, CHUNK)], d16b[b], isem[b])

        def wait_idx(b):
            pltpu.make_async_copy(
                dst_hbm.at[pl.ds(0, CHUNK)], dscb[b], isem[b]).wait()
            pltpu.make_async_copy(
                dst16_hbm.at[pl.ds(0, CHUNK)], d16b[b], isem[b]).wait()

        def issue_scat(b):
            pltpu.async_copy(out_rows, acc_sh.at[dscb[b]], ssem, add=True)
            pltpu.async_copy(den_rows, den_sh.at[d16b[b]], ssem, add=True)

        def wait_scat():
            pltpu.make_async_copy(
                zero_hbm.at[pl.ds(0, CHUNK)], out_rows, ssem).wait()
            pltpu.make_async_copy(
                zero_hbm.at[pl.ds(0, CHUNK)], den_rows, ssem).wait()

        def process(k, b):
            kvr = kvb[b]
            qr = qb[b]
            off = k * CHUNK

            def one_edge(e):
                sc = jnp.zeros((16,), jnp.float32)
                for h in range(HEADS):
                    kvec = kvr[e, pl.ds(h * HD, HD)]
                    qvec = qr[e, pl.ds(h * HD, HD)]
                    splat = allsum16(kvec * qvec) * 0.25
                    eh = jnp.exp(jnp.clip(splat, -5.0, 5.0))
                    vvec = kvr[e, pl.ds(D + h * HD, HD)]
                    out_rows[e, pl.ds(h * HD, HD)] = eh * vvec
                    sc = jnp.where(lane_ids == h, eh, sc)
                dvec = dstv_big[pl.ds(off + e, 16)]
                dsplat = dvec.at[zeros16i].get(mode='promise_in_bounds')
                slotm8 = (dsplat & 15) - lane_ge8
                sc2 = sc.at[lane_mod8].get(mode='promise_in_bounds')
                for j in range(HEADS):
                    dj = jnp.where(slotm8 == 2 * j, sc2, 0.0)
                    den_rows[e, pl.ds(j * 16, 16)] = dj

            def edge(i, carry2):
                e = i * 4
                one_edge(e)
                one_edge(e + 1)
                one_edge(e + 2)
                one_edge(e + 3)
                return carry2

            lax.fori_loop(0, CHUNK // 4, edge, 0)

        def blk(k, b):
            @pl.when(k + 1 < KB)
            def _():
                grab(k + 1, 1 - b)
            wait_grab(b)
            if b == 0:
                @pl.when(k > 0)
                def _():
                    wait_scat()
            else:
                wait_scat()
            @pl.when(k + 1 < KB)
            def _():
                grab_idx(k + 1, 1 - b)
            process(k, b)
            if b == 0:
                @pl.when(k > 0)
                def _():
                    wait_idx(0)
            else:
                wait_idx(1)
            issue_scat(b)

        pltpu.sync_copy(dst_hbm.at[pl.ds(base, CHUNK)], dsc0)
        pltpu.sync_copy(dst16_hbm.at[pl.ds(base, CHUNK)], d16sc0)
        grab(0, 0)

        def pair(g, carry2):
            blk(g * 2, 0)
            blk(g * 2 + 1, 1)
            return carry2

        lax.fori_loop(0, KB // 2, pair, 0)
        wait_scat()
        return carry

    lax.fori_loop(0, NSB, sblk, 0)

    plsc.subcore_barrier()

    @pl.when(s == 0)
    def _():
        pltpu.sync_copy(acc_sh, num_hbm.at[c])
        pltpu.sync_copy(den_sh, den_hbm.at[c])


_edge_kernel = functools.partial(
    pl.kernel,
    mesh=plsc.VectorSubcoreMesh(core_axis_name="c", subcore_axis_name="s"),
    compiler_params=pltpu.CompilerParams(needs_layout_passes=False),
    out_type=[
        jax.ShapeDtypeStruct((2, N, D), jnp.float32),
        jax.ShapeDtypeStruct((2, NDEN, D), jnp.float32),
    ],
    scratch_types=[
        pltpu.VMEM((SB,), jnp.int32),
        pltpu.VMEM((SB + 16,), jnp.int32),
        pltpu.VMEM((CHUNK,), jnp.int32),
        pltpu.VMEM((CHUNK,), jnp.int32),
        pltpu.VMEM((CHUNK,), jnp.int32),
        pltpu.VMEM((CHUNK,), jnp.int32),
        pltpu.VMEM((CHUNK, 2 * D), jnp.float32),
        pltpu.VMEM((CHUNK, 2 * D), jnp.float32),
        pltpu.VMEM((CHUNK, D), jnp.float32),
        pltpu.VMEM((CHUNK, D), jnp.float32),
        pltpu.VMEM((CHUNK, D), jnp.float32),
        pltpu.VMEM((CHUNK, D), jnp.float32),
        pltpu.VMEM_SHARED((N, D), jnp.float32),
        pltpu.VMEM_SHARED((NDEN, D), jnp.float32),
        pltpu.SemaphoreType.DMA,
        pltpu.SemaphoreType.DMA,
        pltpu.SemaphoreType.DMA,
        pltpu.SemaphoreType.DMA,
        pltpu.SemaphoreType.DMA,
    ],
)(_edge_body)


def kernel(x_drug, x_disease, edge_index_drug, edge_index_disease, params):
    expand = (
        (lax.iota(jnp.int32, HEADS)[:, None]
         == (lax.iota(jnp.int32, D)[None, :] // HD))
        .astype(jnp.float32)
    )
    zeros_acc = jnp.zeros((N, D), jnp.float32)

    outs = []
    for x, ei, wk, bk in (
        (x_drug, edge_index_drug, 'W_h_drug', 'b_h_drug'),
        (x_disease, edge_index_disease, 'W_h_disease', 'b_h_disease'),
    ):
        src = ei[0].astype(jnp.int32)
        dst = ei[1].astype(jnp.int32)
        dst16 = dst // 16
        h = _proj(x, params[wk], params[bk])
        for lp in params['layers']:
            wkv = jnp.concatenate([lp['WK'], lp['WV']], axis=1)
            kv, q = _qkv(h, wkv, lp['WQ'])
            num, den = _edge_kernel(kv, q, src, dst, dst16, zeros_acc)
            den = den.reshape(2, NDEN * 16, HEADS)[:, :N]
            h = _post(num, den, h, lp, expand)
        outs.append(h)
    return jnp.stack(outs)


# speedup vs baseline: 2.2787x; 2.2787x over previous
import functools

import jax
import jax.numpy as jnp
from jax import lax
from jax.experimental import pallas as pl
from jax.experimental.pallas import tpu as pltpu
from jax.experimental.pallas import tpu_sc as plsc

N = 10000
D = 128
E = 320000
HEADS = 8
HD = 16
NW = 32
EPW = E // NW
CHUNK = 40
NBLK = EPW // CHUNK
NDEN = 632
SB = 400
KB = SB // CHUNK
NSB = EPW // SB


_BM = 1000


def _proj_body(x_ref, w_ref, b_ref, o_ref):
    o_ref[...] = (
        jnp.dot(x_ref[...], w_ref[...], preferred_element_type=jnp.float32)
        + b_ref[...]
    )


def _proj(x, w, b):
    return pl.pallas_call(
        _proj_body,
        grid=(N // _BM,),
        in_specs=[
            pl.BlockSpec((_BM, D), lambda i: (i, 0)),
            pl.BlockSpec((D, D), lambda i: (0, 0)),
            pl.BlockSpec((1, D), lambda i: (0, 0)),
        ],
        out_specs=pl.BlockSpec((_BM, D), lambda i: (i, 0)),
        out_shape=jax.ShapeDtypeStruct((N, D), jnp.float32),
    )(x, w, b.reshape(1, D))


def _qkv_body(h_ref, wkv_ref, wq_ref, kv_ref, q_ref):
    h = h_ref[...]
    kv_ref[...] = jnp.dot(h, wkv_ref[...], preferred_element_type=jnp.float32)
    q_ref[...] = jnp.dot(h, wq_ref[...], preferred_element_type=jnp.float32)


def _qkv(h, wkv, wq):
    return pl.pallas_call(
        _qkv_body,
        grid=(N // _BM,),
        in_specs=[
            pl.BlockSpec((_BM, D), lambda i: (i, 0)),
            pl.BlockSpec((D, 2 * D), lambda i: (0, 0)),
            pl.BlockSpec((D, D), lambda i: (0, 0)),
        ],
        out_specs=[
            pl.BlockSpec((_BM, 2 * D), lambda i: (i, 0)),
            pl.BlockSpec((_BM, D), lambda i: (i, 0)),
        ],
        out_shape=[
            jax.ShapeDtypeStruct((N, 2 * D), jnp.float32),
            jax.ShapeDtypeStruct((N, D), jnp.float32),
        ],
    )(h, wkv, wq)


def _layer_norm(x, g, b):
    mu = jnp.mean(x, axis=-1, keepdims=True)
    xc = x - mu
    var = jnp.mean(xc * xc, axis=-1, keepdims=True)
    return xc * lax.rsqrt(var + 1e-5) * g + b


def _post_body(num_ref, den_ref, h_ref, exp_ref, wo_ref, bo_ref, g1_ref,
               b1_ref, w1_ref, bf1_ref, w2_ref, bf2_ref, g2_ref, b2_ref,
               o_ref):
    numer = num_ref[0] + num_ref[1]
    den8 = den_ref[0] + den_ref[1]
    dd = jnp.dot(den8, exp_ref[...], preferred_element_type=jnp.float32)
    att = numer / (dd + 1e-6)
    y = (
        jnp.dot(att, wo_ref[...], preferred_element_type=jnp.float32)
        + bo_ref[...]
        + h_ref[...]
    )
    y = _layer_norm(y, g1_ref[...], b1_ref[...])
    f = jnp.maximum(
        jnp.dot(y, w1_ref[...], preferred_element_type=jnp.float32)
        + bf1_ref[...],
        0.0,
    )
    f = jnp.dot(f, w2_ref[...], preferred_element_type=jnp.float32) + bf2_ref[...]
    z = y + f
    o_ref[...] = _layer_norm(z, g2_ref[...], b2_ref[...])


def _post(num, den, h, lp, expand):
    row = lambda v: v.reshape(1, -1)
    full = lambda shp: pl.BlockSpec(shp, lambda i: (0,) * len(shp))
    return pl.pallas_call(
        _post_body,
        grid=(N // _BM,),
        in_specs=[
            pl.BlockSpec((2, _BM, D), lambda i: (0, i, 0)),
            pl.BlockSpec((2, _BM, HEADS), lambda i: (0, i, 0)),
            pl.BlockSpec((_BM, D), lambda i: (i, 0)),
            full((HEADS, D)),
            full((D, D)),
            full((1, D)),
            full((1, D)),
            full((1, D)),
            full((D, 2 * D)),
            full((1, 2 * D)),
            full((2 * D, D)),
            full((1, D)),
            full((1, D)),
            full((1, D)),
        ],
        out_specs=pl.BlockSpec((_BM, D), lambda i: (i, 0)),
        out_shape=jax.ShapeDtypeStruct((N, D), jnp.float32),
    )(
        num, den, h, expand, lp['WO'], row(lp['bO']), row(lp['ln1_g']),
        row(lp['ln1_b']), lp['W1'], row(lp['b1']), lp['W2'], row(lp['b2']),
        row(lp['ln2_g']), row(lp['ln2_b']),
    )


def _edge_body(kv_hbm, q_hbm, src_hbm, dst_hbm, dst16_hbm,
               zero_hbm, num_hbm, den_hbm,
               src_big, dstv_big, dsc0, dsc1, d16sc0, d16sc1,
               kv0, kv1, q0, q1, out_rows, den_rows, acc_sh, den_sh,
               gsem0, gsem1, ssem, isem0, isem1):
    c = lax.axis_index("c")
    s = lax.axis_index("s")

    @pl.when(s == 0)
    def _():
        pltpu.sync_copy(zero_hbm, acc_sh)
        pltpu.sync_copy(zero_hbm.at[pl.ds(0, NDEN)], den_sh)

    plsc.subcore_barrier()

    lane_ids = lax.iota(jnp.int32, 16)
    lane_ge8 = (lane_ids >= 8).astype(jnp.int32)
    zeros16i = jnp.zeros((16,), jnp.int32)
    idx8 = lane_ids ^ 8
    idx4 = lane_ids ^ 4
    idx2 = lane_ids ^ 2
    idx1 = lane_ids ^ 1
    half_lo = lane_ids < 8
    bit2_lo = (lane_ids & 4) == 0
    bit1_lo = (lane_ids & 2) == 0
    lane_of = [2 * (((h & 1) << 2) | (h & 2) | ((h & 4) >> 2))
               for h in range(HEADS)]
    perm = (((lane_ids & 1) << 3) | ((lane_ids & 2) << 1)
            | ((lane_ids & 4) >> 1))
    permh = [zeros16i + lane_of[h] for h in range(HEADS)]

    kvb = (kv0, kv1)
    qb = (q0, q1)
    dscb = (dsc0, dsc1)
    d16b = (d16sc0, d16sc1)
    gsem = (gsem0, gsem1)
    isem = (isem0, isem1)
    wid = c * 16 + s
    base_w = wid * EPW

    def sblk(sb, carry):
        base = base_w + sb * SB
        pltpu.sync_copy(src_hbm.at[pl.ds(base, SB)], src_big)
        pltpu.sync_copy(dst_hbm.at[pl.ds(base, SB)], dstv_big.at[pl.ds(0, SB)])

        def grab(k, b):
            off = pl.multiple_of(k * CHUNK, 8)
            for t in range(0, CHUNK, 8):
                i1 = src_big.at[pl.ds(off + t, 8)]
                i2 = dstv_big.at[pl.ds(off + t, 8)]
                pltpu.async_copy(kv_hbm.at[i1], kvb[b].at[pl.ds(t, 8)],
                                 gsem[b])
                pltpu.async_copy(q_hbm.at[i2], qb[b].at[pl.ds(t, 8)],
                                 gsem[b])

        def wait_grab(b):
            pltpu.make_async_copy(
                kv_hbm.at[pl.ds(0, CHUNK)], kvb[b], gsem[b]).wait()
            pltpu.make_async_copy(
                q_hbm.at[pl.ds(0, CHUNK)], qb[b], gsem[b]).wait()

        def grab_idx(k, b):
            b0 = base + k * CHUNK
            pltpu.async_copy(dst_hbm.at[pl.ds(b0, CHUNK)], dscb[b], isem[b])
            pltpu.async_copy(dst16_hbm.at[pl.ds(b0, CHUNK)], d16b[b], isem[b])

        def wait_idx(b):
            pltpu.make_async_copy(
                dst_hbm.at[pl.ds(0, CHUNK)], dscb[b], isem[b]).wait()
            pltpu.make_async_copy(
                dst16_hbm.at[pl.ds(0, CHUNK)], d16b[b], isem[b]).wait()

        def issue_scat(b):
            pltpu.async_copy(out_rows, acc_sh.at[dscb[b]], ssem, add=True)
            pltpu.async_copy(den_rows, den_sh.at[d16b[b]], ssem, add=True)

        def wait_scat():
            pltpu.make_async_copy(
                zero_hbm.at[pl.ds(0, CHUNK)], out_rows, ssem).wait()
            pltpu.make_async_copy(
                zero_hbm.at[pl.ds(0, CHUNK)], den_rows, ssem).wait()

        def process(k, b):
            kvr = kvb[b]
            qr = qb[b]
            off = k * CHUNK

            def one_edge(e):
                g = lambda v, i: v.at[i].get(mode='promise_in_bounds')
                p = []
                for h in range(HEADS):
                    kvec = kvr[e, pl.ds(h * HD, HD)]
                    qvec = qr[e, pl.ds(h * HD, HD)]
                    t = kvec * qvec
                    p.append(t + g(t, idx8))
                m = []
                for a in range(4):
                    t = jnp.where(half_lo, p[2 * a], p[2 * a + 1])
                    m.append(t + g(t, idx4))
                mm = []
                for a in range(2):
                    t = jnp.where(bit2_lo, m[2 * a], m[2 * a + 1])
                    mm.append(t + g(t, idx2))
                t = jnp.where(bit1_lo, mm[0], mm[1])
                f = t + g(t, idx1)
                ex = jnp.exp(jnp.clip(f, -5.0, 5.0))
                for h in range(HEADS):
                    vvec = kvr[e, pl.ds(D + h * HD, HD)]
                    out_rows[e, pl.ds(h * HD, HD)] = g(ex, permh[h]) * vvec
                sc2 = g(ex, perm)
                dvec = dstv_big[pl.ds(off + e, 16)]
                dsplat = g(dvec, zeros16i)
                slotm8 = (dsplat & 15) - lane_ge8
                for j in range(HEADS):
                    dj = jnp.where(slotm8 == 2 * j, sc2, 0.0)
                    den_rows[e, pl.ds(j * 16, 16)] = dj

            def edge(i, carry2):
                e = i * 4
                one_edge(e)
                one_edge(e + 1)
                one_edge(e + 2)
                one_edge(e + 3)
                return carry2

            lax.fori_loop(0, CHUNK // 4, edge, 0)

        def blk(k, b):
            @pl.when(k + 1 < KB)
            def _():
                grab(k + 1, 1 - b)
            wait_grab(b)
            if b == 0:
                @pl.when(k > 0)
                def _():
                    wait_scat()
            else:
                wait_scat()
            @pl.when(k + 1 < KB)
            def _():
                grab_idx(k + 1, 1 - b)
            process(k, b)
            if b == 0:
                @pl.when(k > 0)
                def _():
                    wait_idx(0)
            else:
                wait_idx(1)
            issue_scat(b)

        pltpu.sync_copy(dst_hbm.at[pl.ds(base, CHUNK)], dsc0)
        pltpu.sync_copy(dst16_hbm.at[pl.ds(base, CHUNK)], d16sc0)
        grab(0, 0)

        def pair(g, carry2):
            blk(g * 2, 0)
            blk(g * 2 + 1, 1)
            return carry2

        lax.fori_loop(0, KB // 2, pair, 0)
        wait_scat()
        return carry

    lax.fori_loop(0, NSB, sblk, 0)

    plsc.subcore_barrier()

    @pl.when(s == 0)
    def _():
        pltpu.sync_copy(acc_sh, num_hbm.at[c])
        pltpu.sync_copy(den_sh, den_hbm.at[c])


_edge_kernel = functools.partial(
    pl.kernel,
    mesh=plsc.VectorSubcoreMesh(core_axis_name="c", subcore_axis_name="s"),
    compiler_params=pltpu.CompilerParams(needs_layout_passes=False),
    out_type=[
        jax.ShapeDtypeStruct((2, N, D), jnp.float32),
        jax.ShapeDtypeStruct((2, NDEN, D), jnp.float32),
    ],
    scratch_types=[
        pltpu.VMEM((SB,), jnp.int32),
        pltpu.VMEM((SB + 16,), jnp.int32),
        pltpu.VMEM((CHUNK,), jnp.int32),
        pltpu.VMEM((CHUNK,), jnp.int32),
        pltpu.VMEM((CHUNK,), jnp.int32),
        pltpu.VMEM((CHUNK,), jnp.int32),
        pltpu.VMEM((CHUNK, 2 * D), jnp.float32),
        pltpu.VMEM((CHUNK, 2 * D), jnp.float32),
        pltpu.VMEM((CHUNK, D), jnp.float32),
        pltpu.VMEM((CHUNK, D), jnp.float32),
        pltpu.VMEM((CHUNK, D), jnp.float32),
        pltpu.VMEM((CHUNK, D), jnp.float32),
        pltpu.VMEM_SHARED((N, D), jnp.float32),
        pltpu.VMEM_SHARED((NDEN, D), jnp.float32),
        pltpu.SemaphoreType.DMA,
        pltpu.SemaphoreType.DMA,
        pltpu.SemaphoreType.DMA,
        pltpu.SemaphoreType.DMA,
        pltpu.SemaphoreType.DMA,
    ],
)(_edge_body)


def kernel(x_drug, x_disease, edge_index_drug, edge_index_disease, params):
    expand = (
        (lax.iota(jnp.int32, HEADS)[:, None]
         == (lax.iota(jnp.int32, D)[None, :] // HD))
        .astype(jnp.float32)
    )
    zeros_acc = jnp.zeros((N, D), jnp.float32)

    outs = []
    for x, ei, wk, bk in (
        (x_drug, edge_index_drug, 'W_h_drug', 'b_h_drug'),
        (x_disease, edge_index_disease, 'W_h_disease', 'b_h_disease'),
    ):
        src = ei[0].astype(jnp.int32)
        dst = ei[1].astype(jnp.int32)
        dst16 = dst // 16
        h = _proj(x, params[wk], params[bk])
        for lp in params['layers']:
            wkv = jnp.concatenate([lp['WK'], lp['WV']], axis=1)
            kv, q = _qkv(h, wkv, lp['WQ'] * 0.25)
            num, den = _edge_kernel(kv, q, src, dst, dst16, zeros_acc)
            den = den.reshape(2, NDEN * 16, HEADS)[:, :N]
            h = _post(num, den, h, lp, expand)
        outs.append(h)
    return jnp.stack(outs)


# speedup vs baseline: 2.4519x; 1.0760x over previous
import functools

import jax
import jax.numpy as jnp
from jax import lax
from jax.experimental import pallas as pl
from jax.experimental.pallas import tpu as pltpu
from jax.experimental.pallas import tpu_sc as plsc

N = 10000
D = 128
E = 320000
HEADS = 8
HD = 16
NW = 32
EPW = E // NW
CHUNK = 40
NBLK = EPW // CHUNK
NDEN = 632
SB = 2000
KB = SB // CHUNK
NSB = EPW // SB


_BM = 1000


def _proj_body(x_ref, w_ref, b_ref, o_ref):
    o_ref[...] = (
        jnp.dot(x_ref[...], w_ref[...], preferred_element_type=jnp.float32)
        + b_ref[...]
    )


def _proj(x, w, b):
    return pl.pallas_call(
        _proj_body,
        grid=(N // _BM,),
        in_specs=[
            pl.BlockSpec((_BM, D), lambda i: (i, 0)),
            pl.BlockSpec((D, D), lambda i: (0, 0)),
            pl.BlockSpec((1, D), lambda i: (0, 0)),
        ],
        out_specs=pl.BlockSpec((_BM, D), lambda i: (i, 0)),
        out_shape=jax.ShapeDtypeStruct((N, D), jnp.float32),
    )(x, w, b.reshape(1, D))


def _qkv_body(h_ref, wkv_ref, wq_ref, kv_ref, q_ref):
    h = h_ref[...]
    kv_ref[...] = jnp.dot(h, wkv_ref[...], preferred_element_type=jnp.float32)
    q_ref[...] = jnp.dot(h, wq_ref[...], preferred_element_type=jnp.float32)


def _qkv(h, wkv, wq):
    return pl.pallas_call(
        _qkv_body,
        grid=(N // _BM,),
        in_specs=[
            pl.BlockSpec((_BM, D), lambda i: (i, 0)),
            pl.BlockSpec((D, 2 * D), lambda i: (0, 0)),
            pl.BlockSpec((D, D), lambda i: (0, 0)),
        ],
        out_specs=[
            pl.BlockSpec((_BM, 2 * D), lambda i: (i, 0)),
            pl.BlockSpec((_BM, D), lambda i: (i, 0)),
        ],
        out_shape=[
            jax.ShapeDtypeStruct((N, 2 * D), jnp.float32),
            jax.ShapeDtypeStruct((N, D), jnp.float32),
        ],
    )(h, wkv, wq)


def _layer_norm(x, g, b):
    mu = jnp.mean(x, axis=-1, keepdims=True)
    xc = x - mu
    var = jnp.mean(xc * xc, axis=-1, keepdims=True)
    return xc * lax.rsqrt(var + 1e-5) * g + b


def _post_body(num_ref, den_ref, h_ref, exp_ref, wo_ref, bo_ref, g1_ref,
               b1_ref, w1_ref, bf1_ref, w2_ref, bf2_ref, g2_ref, b2_ref,
               o_ref):
    numer = num_ref[0] + num_ref[1]
    den8 = den_ref[0] + den_ref[1]
    dd = jnp.dot(den8, exp_ref[...], preferred_element_type=jnp.float32)
    att = numer / (dd + 1e-6)
    y = (
        jnp.dot(att, wo_ref[...], preferred_element_type=jnp.float32)
        + bo_ref[...]
        + h_ref[...]
    )
    y = _layer_norm(y, g1_ref[...], b1_ref[...])
    f = jnp.maximum(
        jnp.dot(y, w1_ref[...], preferred_element_type=jnp.float32)
        + bf1_ref[...],
        0.0,
    )
    f = jnp.dot(f, w2_ref[...], preferred_element_type=jnp.float32) + bf2_ref[...]
    z = y + f
    o_ref[...] = _layer_norm(z, g2_ref[...], b2_ref[...])


def _post(num, den, h, lp, expand):
    row = lambda v: v.reshape(1, -1)
    full = lambda shp: pl.BlockSpec(shp, lambda i: (0,) * len(shp))
    return pl.pallas_call(
        _post_body,
        grid=(N // _BM,),
        in_specs=[
            pl.BlockSpec((2, _BM, D), lambda i: (0, i, 0)),
            pl.BlockSpec((2, _BM, HEADS), lambda i: (0, i, 0)),
            pl.BlockSpec((_BM, D), lambda i: (i, 0)),
            full((HEADS, D)),
            full((D, D)),
            full((1, D)),
            full((1, D)),
            full((1, D)),
            full((D, 2 * D)),
            full((1, 2 * D)),
            full((2 * D, D)),
            full((1, D)),
            full((1, D)),
            full((1, D)),
        ],
        out_specs=pl.BlockSpec((_BM, D), lambda i: (i, 0)),
        out_shape=jax.ShapeDtypeStruct((N, D), jnp.float32),
    )(
        num, den, h, expand, lp['WO'], row(lp['bO']), row(lp['ln1_g']),
        row(lp['ln1_b']), lp['W1'], row(lp['b1']), lp['W2'], row(lp['b2']),
        row(lp['ln2_g']), row(lp['ln2_b']),
    )


def _edge_body(kv_hbm, q_hbm, src_hbm, dst_hbm, dst16_hbm,
               zero_hbm, num_hbm, den_hbm,
               src_big, dstv_big, dsc0, dsc1, d16sc0, d16sc1,
               kv0, kv1, q0, q1, out_rows, den_rows, acc_sh, den_sh,
               gsem0, gsem1, ssem, isem0, isem1):
    c = lax.axis_index("c")
    s = lax.axis_index("s")

    @pl.when(s == 0)
    def _():
        pltpu.sync_copy(zero_hbm, acc_sh)
        pltpu.sync_copy(zero_hbm.at[pl.ds(0, NDEN)], den_sh)

    plsc.subcore_barrier()

    lane_ids = lax.iota(jnp.int32, 16)
    lane_ge8 = (lane_ids >= 8).astype(jnp.int32)
    zeros16i = jnp.zeros((16,), jnp.int32)
    idx8 = lane_ids ^ 8
    idx4 = lane_ids ^ 4
    idx2 = lane_ids ^ 2
    idx1 = lane_ids ^ 1
    half_lo = lane_ids < 8
    bit2_lo = (lane_ids & 4) == 0
    bit1_lo = (lane_ids & 2) == 0
    lane_of = [2 * (((h & 1) << 2) | (h & 2) | ((h & 4) >> 2))
               for h in range(HEADS)]
    perm = (((lane_ids & 1) << 3) | ((lane_ids & 2) << 1)
            | ((lane_ids & 4) >> 1))
    permh = [zeros16i + lane_of[h] for h in range(HEADS)]

    kvb = (kv0, kv1)
    qb = (q0, q1)
    dscb = (dsc0, dsc1)
    d16b = (d16sc0, d16sc1)
    gsem = (gsem0, gsem1)
    isem = (isem0, isem1)
    wid = c * 16 + s
    base_w = wid * EPW

    def sblk(sb, carry):
        base = base_w + sb * SB
        pltpu.sync_copy(src_hbm.at[pl.ds(base, SB)], src_big)
        pltpu.sync_copy(dst_hbm.at[pl.ds(base, SB)], dstv_big.at[pl.ds(0, SB)])

        def grab(k, b):
            off = pl.multiple_of(k * CHUNK, 8)
            for t in range(0, CHUNK, 8):
                i1 = src_big.at[pl.ds(off + t, 8)]
                i2 = dstv_big.at[pl.ds(off + t, 8)]
                pltpu.async_copy(kv_hbm.at[i1], kvb[b].at[pl.ds(t, 8)],
                                 gsem[b])
                pltpu.async_copy(q_hbm.at[i2], qb[b].at[pl.ds(t, 8)],
                                 gsem[b])

        def wait_grab(b):
            pltpu.make_async_copy(
                kv_hbm.at[pl.ds(0, CHUNK)], kvb[b], gsem[b]).wait()
            pltpu.make_async_copy(
                q_hbm.at[pl.ds(0, CHUNK)], qb[b], gsem[b]).wait()

        def grab_idx(k, b):
            b0 = base + k * CHUNK
            pltpu.async_copy(dst_hbm.at[pl.ds(b0, CHUNK)], dscb[b], isem[b])
            pltpu.async_copy(dst16_hbm.at[pl.ds(b0, CHUNK)], d16b[b], isem[b])

        def wait_idx(b):
            pltpu.make_async_copy(
                dst_hbm.at[pl.ds(0, CHUNK)], dscb[b], isem[b]).wait()
            pltpu.make_async_copy(
                dst16_hbm.at[pl.ds(0, CHUNK)], d16b[b], isem[b]).wait()

        def issue_scat(b):
            pltpu.async_copy(out_rows, acc_sh.at[dscb[b]], ssem, add=True)
            pltpu.async_copy(den_rows, den_sh.at[d16b[b]], ssem, add=True)

        def wait_scat():
            pltpu.make_async_copy(
                zero_hbm.at[pl.ds(0, CHUNK)], out_rows, ssem).wait()
            pltpu.make_async_copy(
                zero_hbm.at[pl.ds(0, CHUNK)], den_rows, ssem).wait()

        def process(k, b):
            kvr = kvb[b]
            qr = qb[b]
            off = k * CHUNK

            def one_edge(e):
                g = lambda v, i: v.at[i].get(mode='promise_in_bounds')
                p = []
                for h in range(HEADS):
                    kvec = kvr[e, pl.ds(h * HD, HD)]
                    qvec = qr[e, pl.ds(h * HD, HD)]
                    t = kvec * qvec
                    p.append(t + g(t, idx8))
                m = []
                for a in range(4):
                    t = jnp.where(half_lo, p[2 * a], p[2 * a + 1])
                    m.append(t + g(t, idx4))
                mm = []
                for a in range(2):
                    t = jnp.where(bit2_lo, m[2 * a], m[2 * a + 1])
                    mm.append(t + g(t, idx2))
                t = jnp.where(bit1_lo, mm[0], mm[1])
                f = t + g(t, idx1)
                ex = jnp.exp(jnp.clip(f, -5.0, 5.0))
                for h in range(HEADS):
                    vvec = kvr[e, pl.ds(D + h * HD, HD)]
                    out_rows[e, pl.ds(h * HD, HD)] = g(ex, permh[h]) * vvec
                sc2 = g(ex, perm)
                dvec = dstv_big[pl.ds(off + e, 16)]
                dsplat = g(dvec, zeros16i)
                slotm8 = (dsplat & 15) - lane_ge8
                for j in range(HEADS):
                    dj = jnp.where(slotm8 == 2 * j, sc2, 0.0)
                    den_rows[e, pl.ds(j * 16, 16)] = dj

            def edge(i, carry2):
                e = i * 4
                one_edge(e)
                one_edge(e + 1)
                one_edge(e + 2)
                one_edge(e + 3)
                return carry2

            lax.fori_loop(0, CHUNK // 4, edge, 0)

        def blk(k, b):
            @pl.when(k + 1 < KB)
            def _():
                grab(k + 1, 1 - b)
            wait_grab(b)
            if b == 0:
                @pl.when(k > 0)
                def _():
                    wait_scat()
            else:
                wait_scat()
            @pl.when(k + 1 < KB)
            def _():
                grab_idx(k + 1, 1 - b)
            process(k, b)
            if b == 0:
                @pl.when(k > 0)
                def _():
                    wait_idx(0)
            else:
                wait_idx(1)
            issue_scat(b)

        pltpu.sync_copy(dst_hbm.at[pl.ds(base, CHUNK)], dsc0)
        pltpu.sync_copy(dst16_hbm.at[pl.ds(base, CHUNK)], d16sc0)
        grab(0, 0)

        def pair(g, carry2):
            blk(g * 2, 0)
            blk(g * 2 + 1, 1)
            return carry2

        lax.fori_loop(0, KB // 2, pair, 0)
        wait_scat()
        return carry

    lax.fori_loop(0, NSB, sblk, 0)

    plsc.subcore_barrier()

    @pl.when(s == 0)
    def _():
        pltpu.sync_copy(acc_sh, num_hbm.at[c])
        pltpu.sync_copy(den_sh, den_hbm.at[c])


_edge_kernel = functools.partial(
    pl.kernel,
    mesh=plsc.VectorSubcoreMesh(core_axis_name="c", subcore_axis_name="s"),
    compiler_params=pltpu.CompilerParams(needs_layout_passes=False),
    out_type=[
        jax.ShapeDtypeStruct((2, N, D), jnp.float32),
        jax.ShapeDtypeStruct((2, NDEN, D), jnp.float32),
    ],
    scratch_types=[
        pltpu.VMEM((SB,), jnp.int32),
        pltpu.VMEM((SB + 16,), jnp.int32),
        pltpu.VMEM((CHUNK,), jnp.int32),
        pltpu.VMEM((CHUNK,), jnp.int32),
        pltpu.VMEM((CHUNK,), jnp.int32),
        pltpu.VMEM((CHUNK,), jnp.int32),
        pltpu.VMEM((CHUNK, 2 * D), jnp.float32),
        pltpu.VMEM((CHUNK, 2 * D), jnp.float32),
        pltpu.VMEM((CHUNK, D), jnp.float32),
        pltpu.VMEM((CHUNK, D), jnp.float32),
        pltpu.VMEM((CHUNK, D), jnp.float32),
        pltpu.VMEM((CHUNK, D), jnp.float32),
        pltpu.VMEM_SHARED((N, D), jnp.float32),
        pltpu.VMEM_SHARED((NDEN, D), jnp.float32),
        pltpu.SemaphoreType.DMA,
        pltpu.SemaphoreType.DMA,
        pltpu.SemaphoreType.DMA,
        pltpu.SemaphoreType.DMA,
        pltpu.SemaphoreType.DMA,
    ],
)(_edge_body)


def kernel(x_drug, x_disease, edge_index_drug, edge_index_disease, params):
    expand = (
        (lax.iota(jnp.int32, HEADS)[:, None]
         == (lax.iota(jnp.int32, D)[None, :] // HD))
        .astype(jnp.float32)
    )
    zeros_acc = jnp.zeros((N, D), jnp.float32)

    outs = []
    for x, ei, wk, bk in (
        (x_drug, edge_index_drug, 'W_h_drug', 'b_h_drug'),
        (x_disease, edge_index_disease, 'W_h_disease', 'b_h_disease'),
    ):
        src = ei[0].astype(jnp.int32)
        dst = ei[1].astype(jnp.int32)
        dst16 = dst // 16
        h = _proj(x, params[wk], params[bk])
        for lp in params['layers']:
            wkv = jnp.concatenate([lp['WK'], lp['WV']], axis=1)
            kv, q = _qkv(h, wkv, lp['WQ'] * 0.25)
            num, den = _edge_kernel(kv, q, src, dst, dst16, zeros_acc)
            den = den.reshape(2, NDEN * 16, HEADS)[:, :N]
            h = _post(num, den, h, lp, expand)
        outs.append(h)
    return jnp.stack(outs)
